# baseline (speedup 1.0000x reference)
import functools

import jax
import jax.numpy as jnp
from jax import lax
from jax.experimental import pallas as pl
from jax.experimental.pallas import tpu as pltpu
from jax.experimental.pallas import tpu_sc as plsc

EMBED_DIM = 64
PACK_W = 2 * EMBED_DIM
CHUNK = 160
NBUF = 2
LANES = 16


@functools.lru_cache(maxsize=None)
def _make_gather(n_tokens: int, vocab: int):
    info = plsc.get_sparse_core_info()
    nc, ns = info.num_cores, info.num_subcores
    nw = nc * ns
    assert n_tokens % (nw * CHUNK) == 0 and CHUNK % LANES == 0
    b_per_w = n_tokens // nw
    n_chunks = b_per_w // CHUNK
    assert n_chunks % NBUF == 0
    n_groups = CHUNK // LANES
    mesh = plsc.VectorSubcoreMesh(core_axis_name="c", subcore_axis_name="s")

    @functools.partial(
        pl.kernel,
        mesh=mesh,
        out_type=jax.ShapeDtypeStruct((n_tokens, EMBED_DIM), jnp.float32),
        scratch_types=[
            pltpu.VMEM((b_per_w,), jnp.int32),
            pltpu.VMEM((b_per_w,), jnp.int32),
            pltpu.VMEM((NBUF, CHUNK, PACK_W), jnp.float32),
            pltpu.VMEM((NBUF, CHUNK, EMBED_DIM), jnp.float32),
            pltpu.SemaphoreType.DMA((NBUF,)),
            pltpu.SemaphoreType.DMA((NBUF,)),
        ],
        compiler_params=pltpu.CompilerParams(needs_layout_passes=False),
    )
    def gather_kernel(slot_hbm, half_hbm, tbl_hbm, out_hbm,
                      slot_v, half_v, rows_v, stage_v, gsem, ssem):
        wid = lax.axis_index("s") * nc + lax.axis_index("c")
        base = wid * b_per_w
        pltpu.sync_copy(slot_hbm.at[pl.ds(base, b_per_w)], slot_v)
        pltpu.sync_copy(half_hbm.at[pl.ds(base, b_per_w)], half_v)
        iota = lax.iota(jnp.int32, LANES)

        def gather_desc(j, b):
            return pltpu.make_async_copy(
                tbl_hbm.at[slot_v.at[pl.ds(j * CHUNK, CHUNK)]],
                rows_v.at[b],
                gsem.at[b],
            )

        def store_desc(j, b):
            return pltpu.make_async_copy(
                stage_v.at[b],
                out_hbm.at[pl.ds(base + j * CHUNK, CHUNK)],
                ssem.at[b],
            )

        def extract(j, b):
            bvec = jnp.full((LANES,), b, jnp.int32)

            def group_body(g, carry):
                s_vec = g * LANES + iota
                off = j * CHUNK + g * LANES
                h_vec = half_v[pl.ds(off, LANES)]
                for w in range(EMBED_DIM):
                    wvec = jnp.full((LANES,), w, jnp.int32)
                    v = plsc.load_gather(rows_v, [bvec, s_vec, h_vec + w])
                    plsc.store_scatter(stage_v, [bvec, s_vec, wvec], v)
                return carry

            lax.fori_loop(0, n_groups, group_body, 0)

        gather_desc(0, 0).start()
        gather_desc(1, 1).start()

        def pair_body(k, carry):
            for b in range(NBUF):
                j = NBUF * k + b
                gather_desc(j, b).wait()

                @pl.when(k >= 1)
                def _():
                    store_desc(j - NBUF, b).wait()

                extract(j, b)
                store_desc(j, b).start()

                @pl.when(k < (n_chunks // NBUF) - 1)
                def _():
                    gather_desc(j + NBUF, b).start()

            return carry

        lax.fori_loop(0, n_chunks // NBUF, pair_body, 0)
        for b in range(NBUF):
            store_desc(n_chunks - NBUF + b, b).wait()

    return gather_kernel


def kernel(token_ids, table):
    b, s = token_ids.shape
    vocab, d = table.shape
    assert d == EMBED_DIM and vocab % 2 == 0
    flat = token_ids.reshape(b * s)
    slot = flat // 2
    half = (flat % 2) * EMBED_DIM
    tblp = table.reshape(vocab // 2, PACK_W)
    out = _make_gather(b * s, vocab)(slot, half, tblp)
    return out.reshape(b, s, EMBED_DIM)


# speedup vs baseline: 1.4586x; 1.4586x over previous
import functools

import jax
import jax.numpy as jnp
from jax import lax
from jax.experimental import pallas as pl
from jax.experimental.pallas import tpu as pltpu
from jax.experimental.pallas import tpu_sc as plsc

EMBED_DIM = 64
CHUNK = 200
NBUF = 4


@functools.lru_cache(maxsize=None)
def _make_gather(bsz: int, seq: int):
    n_tokens = bsz * seq
    info = plsc.get_sparse_core_info()
    nc, ns = info.num_cores, info.num_subcores
    nw = nc * ns
    assert n_tokens % (nw * CHUNK) == 0 and seq == CHUNK
    b_per_w = n_tokens // nw
    n_chunks = b_per_w // CHUNK
    mesh = plsc.VectorSubcoreMesh(core_axis_name="c", subcore_axis_name="s")

    @functools.partial(
        pl.kernel,
        mesh=mesh,
        out_type=jax.ShapeDtypeStruct((bsz, seq, EMBED_DIM), jnp.float32),
        scratch_types=[
            pltpu.VMEM((b_per_w,), jnp.int32),
            pltpu.VMEM((NBUF, CHUNK, EMBED_DIM), jnp.float32),
            pltpu.SemaphoreType.DMA((NBUF,)),
            pltpu.SemaphoreType.DMA((NBUF,)),
        ],
        compiler_params=pltpu.CompilerParams(use_tc_tiling_on_sc=False),
    )
    def gather_kernel(idx_hbm, tbl_hbm, out_hbm, idx_v, rows_v, gsem, ssem):
        wid = lax.axis_index("s") * nc + lax.axis_index("c")
        base = wid * b_per_w
        pltpu.sync_copy(idx_hbm.at[pl.ds(base, b_per_w)], idx_v)

        def issue_gather(j):
            b = j % NBUF
            return pltpu.async_copy(
                tbl_hbm.at[idx_v.at[pl.ds(j * CHUNK, CHUNK)]],
                rows_v.at[b],
                gsem.at[b],
            )

        def issue_store(j):
            b = j % NBUF
            return pltpu.async_copy(
                rows_v.at[b],
                out_hbm.at[wid * n_chunks + j],
                ssem.at[b],
            )

        gd = [None] * n_chunks
        sd = [None] * n_chunks
        for b in range(min(NBUF, n_chunks)):
            gd[b] = issue_gather(b)
        for j in range(n_chunks):
            gd[j].wait()
            sd[j] = issue_store(j)
            jn = j + NBUF
            if jn < n_chunks:
                sd[j].wait()
                gd[jn] = issue_gather(jn)
        for j in range(max(0, n_chunks - NBUF), n_chunks):
            sd[j].wait()

    return gather_kernel


def kernel(token_ids, table):
    b, s = token_ids.shape
    flat_idx = token_ids.reshape(b * s)
    return _make_gather(b, s)(flat_idx, table)


# speedup vs baseline: 1.8063x; 1.2384x over previous
import functools

import jax
import jax.numpy as jnp
from jax import lax
from jax.experimental import pallas as pl
from jax.experimental.pallas import tpu as pltpu
from jax.experimental.pallas import tpu_sc as plsc

EMBED_DIM = 64
PACK_W = 2 * EMBED_DIM
NBUF = 2
PACK_COLS = 2048


@functools.lru_cache(maxsize=None)
def _make_pack(vocab: int):
    grid = pl.cdiv(vocab, PACK_COLS)

    @functools.partial(
        pl.pallas_call,
        grid=(grid,),
        in_specs=[pl.BlockSpec((EMBED_DIM, PACK_COLS), lambda i: (0, i))],
        out_specs=pl.BlockSpec((PACK_COLS, PACK_W), lambda i: (i, 0)),
        out_shape=jax.ShapeDtypeStruct((vocab, PACK_W), jnp.float32),
    )
    def pack_kernel(tin, tout):
        tout[:, 0:EMBED_DIM] = jnp.transpose(tin[...])

    return pack_kernel


@functools.lru_cache(maxsize=None)
def _make_gather(bsz: int, seq: int, vocab: int):
    n_tokens = bsz * seq
    info = plsc.get_sparse_core_info()
    nc, ns = info.num_cores, info.num_subcores
    nw = nc * ns
    b_per_w = n_tokens // nw
    chunk = seq
    n_chunks = b_per_w // chunk
    assert n_tokens % (nw * chunk) == 0
    mesh = plsc.VectorSubcoreMesh(core_axis_name="c", subcore_axis_name="s")

    @functools.partial(
        pl.kernel,
        mesh=mesh,
        out_type=jax.ShapeDtypeStruct((bsz, seq, EMBED_DIM), jnp.float32),
        scratch_types=[
            pltpu.VMEM((b_per_w,), jnp.int32),
            pltpu.VMEM((NBUF, chunk, PACK_W), jnp.float32),
            pltpu.VMEM((NBUF, chunk, EMBED_DIM), jnp.float32),
            pltpu.SemaphoreType.DMA((NBUF,)),
            pltpu.SemaphoreType.DMA((NBUF,)),
        ],
        compiler_params=pltpu.CompilerParams(
            needs_layout_passes=False,
            disable_bounds_checks=True,
        ),
    )
    def gather_kernel(idx_hbm, tbl_hbm, out_hbm,
                      idx_v, rows_v, stage_v, gsem, ssem):
        wid = lax.axis_index("s") * nc + lax.axis_index("c")
        base = wid * b_per_w
        pltpu.sync_copy(idx_hbm.at[pl.ds(base, b_per_w)], idx_v)

        def issue_gather(j):
            b = j % NBUF
            return pltpu.async_copy(
                tbl_hbm.at[idx_v.at[pl.ds(j * chunk, chunk)]],
                rows_v.at[b],
                gsem.at[b],
            )

        def issue_store(j):
            b = j % NBUF
            return pltpu.async_copy(
                stage_v.at[b],
                out_hbm.at[wid * n_chunks + j],
                ssem.at[b],
            )

        gd = [None] * n_chunks
        sd = [None] * n_chunks
        for j in range(min(2, n_chunks)):
            gd[j] = issue_gather(j)
        for j in range(n_chunks):
            b = j % NBUF
            gd[j].wait()
            if j >= NBUF:
                sd[j - NBUF].wait()
            def tok_body(t, carry):
                for q in range(EMBED_DIM // 16):
                    stage_v[b, t, pl.ds(q * 16, 16)] = (
                        rows_v[b, t, pl.ds(q * 16, 16)])
                return carry

            lax.fori_loop(0, chunk, tok_body, 0)
            sd[j] = issue_store(j)
            if j + 2 < n_chunks:
                gd[j + 2] = issue_gather(j + 2)
        for j in range(max(0, n_chunks - NBUF), n_chunks):
            sd[j].wait()

    return gather_kernel


def kernel(token_ids, table):
    b, s = token_ids.shape
    vocab, d = table.shape
    assert d == EMBED_DIM
    flat = token_ids.reshape(b * s)
    packed = _make_pack(vocab)(jnp.swapaxes(table, 0, 1))
    return _make_gather(b, s, vocab)(flat, packed)
